# baseline (speedup 1.0000x reference)
import functools

import jax
import jax.numpy as jnp
from jax import lax
from jax.experimental import pallas as pl
from jax.experimental.pallas import tpu as pltpu
from jax.experimental.pallas import tpu_sc as plsc

NUM_CORES = 2
NUM_SUBCORES = 16
NUM_WORKERS = NUM_CORES * NUM_SUBCORES
CHUNK = 128


@functools.lru_cache(maxsize=None)
def _make_gather(b_total: int, d: int):
    assert b_total % (NUM_WORKERS * CHUNK) == 0
    b_per_w = b_total // NUM_WORKERS
    n_chunks = b_per_w // CHUNK
    assert n_chunks >= 4 and n_chunks % 2 == 0
    mesh = plsc.VectorSubcoreMesh(core_axis_name="c", subcore_axis_name="s")

    @functools.partial(
        pl.kernel,
        mesh=mesh,
        out_type=jax.ShapeDtypeStruct((b_total, d), jnp.float32),
        scratch_types=[
            pltpu.VMEM((n_chunks, CHUNK), jnp.int32),
            pltpu.VMEM((2, CHUNK, d), jnp.float32),
            pltpu.SemaphoreType.DMA,
            pltpu.SemaphoreType.DMA,
            pltpu.SemaphoreType.DMA,
            pltpu.SemaphoreType.DMA,
        ],
        compiler_params=pltpu.CompilerParams(use_tc_tiling_on_sc=False),
    )
    def gather_kernel(x_hbm, table_hbm, out_hbm, idx_v, rows_v, g0, g1, o0, o1):
        wid = lax.axis_index("s") * NUM_CORES + lax.axis_index("c")
        base = wid * b_per_w
        pltpu.sync_copy(x_hbm.at[wid], idx_v)

        gsem = (g0, g1)
        osem = (o0, o1)

        def start_gather(c, b):
            pltpu.async_copy(table_hbm.at[idx_v.at[c]], rows_v.at[b], gsem[b])

        def wait_gather(b):
            pltpu.make_async_copy(
                table_hbm.at[idx_v.at[0]], rows_v.at[b], gsem[b]
            ).wait()

        def start_out(c, b):
            pltpu.async_copy(
                rows_v.at[b], out_hbm.at[pl.ds(base + c * CHUNK, CHUNK)], osem[b]
            )

        def wait_out(b):
            pltpu.make_async_copy(
                rows_v.at[b], out_hbm.at[pl.ds(base, CHUNK)], osem[b]
            ).wait()

        start_gather(0, 0)
        start_gather(1, 1)
        wait_gather(0)
        start_out(0, 0)
        wait_gather(1)
        start_out(1, 1)
        wait_out(0)
        start_gather(2, 0)
        wait_out(1)
        start_gather(3, 1)

        @pl.loop(1, (n_chunks - 2) // 2)
        def _steady(i):
            ci = 2 * i
            wait_gather(0)
            start_out(ci, 0)
            wait_gather(1)
            start_out(ci + 1, 1)
            wait_out(0)
            start_gather(ci + 2, 0)
            wait_out(1)
            start_gather(ci + 3, 1)

        wait_gather(0)
        start_out(n_chunks - 2, 0)
        wait_gather(1)
        start_out(n_chunks - 1, 1)
        wait_out(0)
        wait_out(1)

    return gather_kernel


def kernel(x, table):
    rows, cols = x.shape
    d = table.shape[1]
    b_total = rows * cols
    b_per_w = b_total // NUM_WORKERS
    x_flat = x.reshape(NUM_WORKERS, b_per_w // CHUNK, CHUNK)
    out = _make_gather(b_total, d)(x_flat, table)
    return out.reshape(rows, cols, d)


# speedup vs baseline: 1.0333x; 1.0333x over previous
import functools

import jax
import jax.numpy as jnp
from jax import lax
from jax.experimental import pallas as pl
from jax.experimental.pallas import tpu as pltpu
from jax.experimental.pallas import tpu_sc as plsc

NUM_CORES = 2
NUM_SUBCORES = 16
NUM_WORKERS = NUM_CORES * NUM_SUBCORES
CHUNK = 128
NBUF = 8


@functools.lru_cache(maxsize=None)
def _make_gather(b_total: int, d: int):
    assert b_total % (NUM_WORKERS * CHUNK) == 0
    b_per_w = b_total // NUM_WORKERS
    n_chunks = b_per_w // CHUNK
    assert n_chunks % NBUF == 0 and n_chunks // NBUF >= 2
    mesh = plsc.VectorSubcoreMesh(core_axis_name="c", subcore_axis_name="s")

    @functools.partial(
        pl.kernel,
        mesh=mesh,
        out_type=jax.ShapeDtypeStruct((b_total, d), jnp.float32),
        scratch_types=[
            pltpu.VMEM((n_chunks, CHUNK), jnp.int32),
            pltpu.VMEM((NBUF, CHUNK, d), jnp.float32),
            pltpu.SemaphoreType.DMA((NBUF,)),
            pltpu.SemaphoreType.DMA((NBUF,)),
        ],
        compiler_params=pltpu.CompilerParams(use_tc_tiling_on_sc=False),
    )
    def gather_kernel(x_hbm, table_hbm, out_hbm, idx_v, rows_v, gsem, osem):
        wid = lax.axis_index("s") * NUM_CORES + lax.axis_index("c")
        base = wid * b_per_w
        pltpu.sync_copy(x_hbm.at[wid], idx_v)

        def start_gather(c, b):
            pltpu.async_copy(table_hbm.at[idx_v.at[c]], rows_v.at[b], gsem.at[b])

        def wait_gather(b):
            pltpu.make_async_copy(
                table_hbm.at[idx_v.at[0]], rows_v.at[b], gsem.at[b]
            ).wait()

        def start_out(c, b):
            pltpu.async_copy(
                rows_v.at[b], out_hbm.at[pl.ds(base + c * CHUNK, CHUNK)], osem.at[b]
            )

        def wait_out(b):
            pltpu.make_async_copy(
                rows_v.at[b], out_hbm.at[pl.ds(base, CHUNK)], osem.at[b]
            ).wait()

        for b in range(NBUF):
            start_gather(b, b)

        @pl.loop(0, n_chunks // NBUF - 1)
        def _steady(i):
            ci = i * NBUF
            for b in range(NBUF):
                wait_gather(b)
                start_out(ci + b, b)
            for b in range(NBUF):
                wait_out(b)
                start_gather(ci + NBUF + b, b)

        ci = n_chunks - NBUF
        for b in range(NBUF):
            wait_gather(b)
            start_out(ci + b, b)
        for b in range(NBUF):
            wait_out(b)

    return gather_kernel


def kernel(x, table):
    rows, cols = x.shape
    d = table.shape[1]
    b_total = rows * cols
    b_per_w = b_total // NUM_WORKERS
    x_flat = x.reshape(NUM_WORKERS, b_per_w // CHUNK, CHUNK)
    out = _make_gather(b_total, d)(x_flat, table)
    return out.reshape(rows, cols, d)
